# baseline (speedup 1.0000x reference)
import jax
import jax.numpy as jnp
from jax import lax
from jax.experimental import pallas as pl
from jax.experimental.pallas import tpu as pltpu
from jax.experimental.pallas import tpu_sc as plsc

NUM_ITEMS = 100000
EMBED = 64
BATCH = 4096
HIST = 200
EPS = 1e-5

N = BATCH * HIST
NC = 2
NS = 16
NW = NC * NS
PER_W = N // NW
K = 512
STEPS = PER_W // K
L = 16

_DNUMS = lax.GatherDimensionNumbers(
    offset_dims=(), collapsed_slice_dims=(0,), start_index_map=(0,))


def _perm(v, idx):
    return lax.gather(v, idx, _DNUMS, (1,),
                      mode=lax.GatherScatterMode.PROMISE_IN_BOUNDS)


def _body(x_hbm, table_hbm, gamma_hbm, beta_hbm, out_hbm, mask_hbm,
          idx_v, idx2_v, rows_v, res_v, gam_v, bet_v, mask_v, sem):
    wid = lax.axis_index("s") * NC + lax.axis_index("c")

    pltpu.sync_copy(gamma_hbm, gam_v)
    pltpu.sync_copy(beta_hbm, bet_v)
    gvecs = [gam_v[pl.ds(L * j, L)] for j in range(EMBED // L)]
    bvecs = [bet_v[pl.ds(L * j, L)] for j in range(EMBED // L)]

    ones = jnp.full((L,), 1, jnp.int32)
    zeros = jnp.full((L,), 0, jnp.int32)
    magic = jnp.full((L,), 0x5F3759DF, jnp.int32)
    lane = lax.iota(jnp.int32, L)
    perms = [jnp.reshape((lane + r) % L, (L, 1)) for r in (8, 4, 2, 1)]

    def step(g, carry):
        base = wid * PER_W + g * K
        pltpu.sync_copy(x_hbm.at[pl.ds(base, K)], idx_v)

        @plsc.parallel_loop(0, K // L, 1, unroll=4)
        def mstep(t):
            iv = idx_v[pl.ds(L * t, L)]
            idx2_v[pl.ds(L * t, L)] = lax.shift_right_logical(iv, 1)
            mask_v[pl.ds(L * t, L)] = jnp.where(iv > 0, ones, zeros)

        pltpu.async_copy(table_hbm.at[idx2_v], rows_v, sem).wait()

        @plsc.parallel_loop(0, K, 1, unroll=4)
        def rstep(r):
            ra = lax.div(r, L) * L
            iv = idx_v[pl.ds(ra, L)]
            sel = _perm(iv, jnp.full((L, 1), lax.rem(r, L), jnp.int32))
            parf = lax.convert_element_type(
                lax.bitwise_and(sel, ones), jnp.float32)
            vs = []
            for j in range(EMBED // L):
                lo = rows_v[r, pl.ds(L * j, L)]
                hi = rows_v[r, pl.ds(EMBED + L * j, L)]
                vs.append(lo + parf * (hi - lo))
            s = (vs[0] + vs[1]) + (vs[2] + vs[3])
            q = (vs[0] * vs[0] + vs[1] * vs[1]) + (vs[2] * vs[2] + vs[3] * vs[3])
            for p in perms:
                s = s + _perm(s, p)
                q = q + _perm(q, p)
            mean = s * (1.0 / EMBED)
            var = q * (1.0 / EMBED) - mean * mean
            av = var + EPS
            yi = magic - lax.shift_right_logical(
                lax.bitcast_convert_type(av, jnp.int32), 1)
            y = lax.bitcast_convert_type(yi, jnp.float32)
            half = av * 0.5
            y = y * (1.5 - half * y * y)
            y = y * (1.5 - half * y * y)
            rr = lax.div(r, 2)
            cc = lax.rem(r, 2) * EMBED
            for j in range(EMBED // L):
                res_v[rr, pl.dslice(cc + L * j, L)] = \
                    (vs[j] - mean) * y * gvecs[j] + bvecs[j]

        pltpu.sync_copy(res_v, out_hbm.at[pl.ds(base // 2, K // 2)])
        pltpu.sync_copy(mask_v, mask_hbm.at[pl.ds(base, K)])
        return carry
    lax.fori_loop(0, STEPS, step, 0)


@jax.jit
def _lru_embed(x_flat, table2, gamma, beta):
    mesh = plsc.VectorSubcoreMesh(core_axis_name="c", subcore_axis_name="s")
    out2, mask_i32 = pl.kernel(
        _body,
        out_type=(
            jax.ShapeDtypeStruct((N // 2, 2 * EMBED), jnp.float32),
            jax.ShapeDtypeStruct((N,), jnp.int32),
        ),
        mesh=mesh,
        compiler_params=pltpu.CompilerParams(use_tc_tiling_on_sc=False),
        scratch_types=[
            pltpu.VMEM((K,), jnp.int32),
            pltpu.VMEM((K,), jnp.int32),
            pltpu.VMEM((K, 2 * EMBED), jnp.float32),
            pltpu.VMEM((K // 2, 2 * EMBED), jnp.float32),
            pltpu.VMEM((EMBED,), jnp.float32),
            pltpu.VMEM((EMBED,), jnp.float32),
            pltpu.VMEM((K,), jnp.int32),
            pltpu.SemaphoreType.DMA,
        ],
    )(x_flat, table2, gamma, beta)
    return out2, mask_i32


def kernel(x, table, gamma, beta):
    x_flat = x.reshape(N).astype(jnp.int32)
    table2 = table.reshape(NUM_ITEMS // 2, 2 * EMBED)
    out2, mask_i32 = _lru_embed(x_flat, table2, gamma, beta)
    out = out2.reshape(BATCH, HIST, EMBED)
    mask = mask_i32.reshape(BATCH, HIST).astype(jnp.bool_)
    return out, mask


# speedup vs baseline: 1.1271x; 1.1271x over previous
import jax
import jax.numpy as jnp
from jax import lax
from jax.experimental import pallas as pl
from jax.experimental.pallas import tpu as pltpu
from jax.experimental.pallas import tpu_sc as plsc

NUM_ITEMS = 100000
EMBED = 64
BATCH = 4096
HIST = 200
EPS = 1e-5

N = BATCH * HIST
NC = 2
NS = 16
NW = NC * NS
PER_W = N // NW
K = 512
STEPS = PER_W // K
L = 16

_DNUMS = lax.GatherDimensionNumbers(
    offset_dims=(), collapsed_slice_dims=(0,), start_index_map=(0,))


def _perm(v, idx):
    return lax.gather(v, idx, _DNUMS, (1,),
                      mode=lax.GatherScatterMode.PROMISE_IN_BOUNDS)


def _body(x_hbm, table_hbm, gamma_hbm, beta_hbm, out_hbm, mask_hbm,
          idx0_v, idx1_v, rows0_v, rows1_v, gam_v, bet_v, mask_v,
          sem0, sem1):
    wid = lax.axis_index("s") * NC + lax.axis_index("c")
    wbase = wid * PER_W

    pltpu.sync_copy(gamma_hbm, gam_v)
    pltpu.sync_copy(beta_hbm, bet_v)
    gvecs = [gam_v[pl.ds(L * j, L)] for j in range(EMBED // L)]
    bvecs = [bet_v[pl.ds(L * j, L)] for j in range(EMBED // L)]

    ones = jnp.full((L,), 1, jnp.int32)
    zeros = jnp.full((L,), 0, jnp.int32)
    magic = jnp.full((L,), 0x5F3759DF, jnp.int32)
    lane = lax.iota(jnp.int32, L)
    perms = [jnp.reshape((lane + r) % L, (L, 1)) for r in (8, 4, 2, 1)]

    bufs = ((idx0_v, rows0_v, sem0), (idx1_v, rows1_v, sem1))

    def prefetch(g, idx_v, rows_v, sem):
        pltpu.sync_copy(x_hbm.at[pl.ds(wbase + g * K, K)], idx_v)
        pltpu.async_copy(table_hbm.at[idx_v], rows_v, sem)

    def process(g, idx_v, rows_v, sem):
        base = wbase + g * K
        pltpu.make_async_copy(table_hbm.at[idx_v], rows_v, sem).wait()

        @plsc.parallel_loop(0, K // L, 1, unroll=4)
        def mstep(t):
            iv = idx_v[pl.ds(L * t, L)]
            mask_v[pl.ds(L * t, L)] = jnp.where(iv > 0, ones, zeros)

        @plsc.parallel_loop(0, K, 1, unroll=8)
        def rstep(r):
            vs = [rows_v[r, pl.ds(L * j, L)] for j in range(EMBED // L)]
            s = (vs[0] + vs[1]) + (vs[2] + vs[3])
            q = (vs[0] * vs[0] + vs[1] * vs[1]) + (vs[2] * vs[2] + vs[3] * vs[3])
            for p in perms:
                s = s + _perm(s, p)
                q = q + _perm(q, p)
            mean = s * (1.0 / EMBED)
            var = q * (1.0 / EMBED) - mean * mean
            av = var + EPS
            yi = magic - lax.shift_right_logical(
                lax.bitcast_convert_type(av, jnp.int32), 1)
            y = lax.bitcast_convert_type(yi, jnp.float32)
            y = y * (1.5 - (av * 0.5) * y * y)
            for j in range(EMBED // L):
                rows_v[r, pl.ds(L * j, L)] = (vs[j] - mean) * y * gvecs[j] + bvecs[j]

        pltpu.sync_copy(rows_v, out_hbm.at[pl.ds(base, K)])
        pltpu.sync_copy(mask_v, mask_hbm.at[pl.ds(base, K)])

    prefetch(0, *bufs[0])

    def step(it, carry):
        g0 = it * 2
        prefetch(g0 + 1, *bufs[1])
        process(g0, *bufs[0])

        @pl.when(g0 + 2 < STEPS)
        def _():
            prefetch(g0 + 2, *bufs[0])
        process(g0 + 1, *bufs[1])
        return carry
    lax.fori_loop(0, STEPS // 2, step, 0)


@jax.jit
def _lru_embed(x_flat, table, gamma, beta):
    mesh = plsc.VectorSubcoreMesh(core_axis_name="c", subcore_axis_name="s")
    out_flat, mask_i32 = pl.kernel(
        _body,
        out_type=(
            jax.ShapeDtypeStruct((N, EMBED), jnp.float32),
            jax.ShapeDtypeStruct((N,), jnp.int32),
        ),
        mesh=mesh,
        compiler_params=pltpu.CompilerParams(use_tc_tiling_on_sc=False),
        scratch_types=[
            pltpu.VMEM((K,), jnp.int32),
            pltpu.VMEM((K,), jnp.int32),
            pltpu.VMEM((K, EMBED), jnp.float32),
            pltpu.VMEM((K, EMBED), jnp.float32),
            pltpu.VMEM((EMBED,), jnp.float32),
            pltpu.VMEM((EMBED,), jnp.float32),
            pltpu.VMEM((K,), jnp.int32),
            pltpu.SemaphoreType.DMA,
            pltpu.SemaphoreType.DMA,
        ],
    )(x_flat, table, gamma, beta)
    return out_flat, mask_i32


def kernel(x, table, gamma, beta):
    x_flat = x.reshape(N).astype(jnp.int32)
    out_flat, mask_i32 = _lru_embed(x_flat, table, gamma, beta)
    out = out_flat.reshape(BATCH, HIST, EMBED)
    mask = mask_i32.reshape(BATCH, HIST).astype(jnp.bool_)
    return out, mask


# speedup vs baseline: 1.2830x; 1.1383x over previous
import jax
import jax.numpy as jnp
from jax import lax
from jax.experimental import pallas as pl
from jax.experimental.pallas import tpu as pltpu
from jax.experimental.pallas import tpu_sc as plsc

NUM_ITEMS = 100000
EMBED = 64
BATCH = 4096
HIST = 200
EPS = 1e-5

N = BATCH * HIST
NC = 2
NS = 16
NW = NC * NS
PER_W = N // NW
K = 512
STEPS = PER_W // K
L = 16

_DNUMS = lax.GatherDimensionNumbers(
    offset_dims=(), collapsed_slice_dims=(0,), start_index_map=(0,))


def _perm(v, idx):
    return lax.gather(v, idx, _DNUMS, (1,),
                      mode=lax.GatherScatterMode.PROMISE_IN_BOUNDS)


def _body(x_hbm, table_hbm, gamma_hbm, beta_hbm, out_hbm, mask_hbm,
          idx0_v, idx1_v, rows0_v, rows1_v, gam_v, bet_v, mask_v,
          sem0, sem1):
    wid = lax.axis_index("s") * NC + lax.axis_index("c")
    wbase = wid * PER_W

    pltpu.sync_copy(gamma_hbm, gam_v)
    pltpu.sync_copy(beta_hbm, bet_v)
    gvecs = [gam_v[pl.ds(L * j, L)] for j in range(EMBED // L)]
    bvecs = [bet_v[pl.ds(L * j, L)] for j in range(EMBED // L)]

    ones = jnp.full((L,), 1, jnp.int32)
    zeros = jnp.full((L,), 0, jnp.int32)
    magic = jnp.full((L,), 0x5F3759DF, jnp.int32)
    lane = lax.iota(jnp.int32, L)
    perms = [jnp.reshape((lane + r) % L, (L, 1)) for r in (8, 4, 2, 1)]

    bufs = ((idx0_v, rows0_v, sem0), (idx1_v, rows1_v, sem1))

    def prefetch(g, idx_v, rows_v, sem):
        pltpu.sync_copy(x_hbm.at[pl.ds(wbase + g * K, K)], idx_v)
        pltpu.async_copy(table_hbm.at[idx_v], rows_v, sem)

    def process(g, idx_v, rows_v, sem):
        base = wbase + g * K
        pltpu.make_async_copy(table_hbm.at[idx_v], rows_v, sem).wait()

        @plsc.parallel_loop(0, K // L, 1, unroll=4)
        def mstep(t):
            iv = idx_v[pl.ds(L * t, L)]
            mask_v[pl.ds(L * t, L)] = jnp.where(iv > 0, ones, zeros)

        @plsc.parallel_loop(0, K, 1, unroll=4)
        def rstep(r):
            vs = [rows_v[r, pl.ds(L * j, L)] for j in range(EMBED // L)]
            s = (vs[0] + vs[1]) + (vs[2] + vs[3])
            q = (vs[0] * vs[0] + vs[1] * vs[1]) + (vs[2] * vs[2] + vs[3] * vs[3])
            for p in perms:
                s = s + _perm(s, p)
                q = q + _perm(q, p)
            mean = s * (1.0 / EMBED)
            var = q * (1.0 / EMBED) - mean * mean
            av = var + EPS
            yi = magic - lax.shift_right_logical(
                lax.bitcast_convert_type(av, jnp.int32), 1)
            y = lax.bitcast_convert_type(yi, jnp.float32)
            y = y * (1.5 - (av * 0.5) * y * y)
            for j in range(EMBED // L):
                rows_v[r, pl.ds(L * j, L)] = (vs[j] - mean) * y * gvecs[j] + bvecs[j]

        pltpu.sync_copy(rows_v, out_hbm.at[pl.ds(base, K)])
        pltpu.sync_copy(mask_v, mask_hbm.at[pl.ds(base, K)])

    prefetch(0, *bufs[0])

    def step(it, carry):
        g0 = it * 2
        prefetch(g0 + 1, *bufs[1])
        process(g0, *bufs[0])

        @pl.when(g0 + 2 < STEPS)
        def _():
            prefetch(g0 + 2, *bufs[0])
        process(g0 + 1, *bufs[1])
        return carry
    lax.fori_loop(0, STEPS // 2, step, 0)


@jax.jit
def _lru_embed(x_flat, table, gamma, beta):
    mesh = plsc.VectorSubcoreMesh(core_axis_name="c", subcore_axis_name="s")
    out_flat, mask_i32 = pl.kernel(
        _body,
        out_type=(
            jax.ShapeDtypeStruct((N, EMBED), jnp.float32),
            jax.ShapeDtypeStruct((N,), jnp.int32),
        ),
        mesh=mesh,
        compiler_params=pltpu.CompilerParams(use_tc_tiling_on_sc=False),
        scratch_types=[
            pltpu.VMEM((K,), jnp.int32),
            pltpu.VMEM((K,), jnp.int32),
            pltpu.VMEM((K, EMBED), jnp.float32),
            pltpu.VMEM((K, EMBED), jnp.float32),
            pltpu.VMEM((EMBED,), jnp.float32),
            pltpu.VMEM((EMBED,), jnp.float32),
            pltpu.VMEM((K,), jnp.int32),
            pltpu.SemaphoreType.DMA,
            pltpu.SemaphoreType.DMA,
        ],
    )(x_flat, table, gamma, beta)
    return out_flat, mask_i32


def kernel(x, table, gamma, beta):
    x_flat = x.reshape(N).astype(jnp.int32)
    out_flat, mask_i32 = _lru_embed(x_flat, table, gamma, beta)
    out = out_flat.reshape(BATCH, HIST, EMBED)
    mask = mask_i32.reshape(BATCH, HIST).astype(jnp.bool_)
    return out, mask
